# Initial kernel scaffold; baseline (speedup 1.0000x reference)
import jax
import jax.numpy as jnp
from jax.experimental import pallas as pl


def kernel(x, tables, W1, b1, gamma1, beta1, W2, b2, gamma2, beta2, W3, b3):
    raise NotImplementedError("write your pallas kernel here")


# speedup vs baseline: 1.9810x; 1.9810x over previous
import functools

import jax
import jax.numpy as jnp
from jax import lax
from jax.experimental import pallas as pl
from jax.experimental.pallas import tpu as pltpu
from jax.experimental.pallas import tpu_sc as plsc

V = 100000
NF = 26
D = 32
B = 16384
EM = NF * D
H1 = 128
H2 = 64
ROWS = B * NF
NW = 32
RPW = ROWS // NW
CHUNK = 128
NBUF = 8
GROUP = CHUNK * NBUF
NGROUP = RPW // GROUP
NPER = 13
PERIOD = NPER * 16
EPS = 1e-5
BT = 1024
T = B // BT


def _sc_gather(xf, tf):
    mesh = plsc.VectorSubcoreMesh(core_axis_name="c", subcore_axis_name="s")

    @functools.partial(
        pl.kernel,
        mesh=mesh,
        out_type=jax.ShapeDtypeStruct((ROWS, D), jnp.float32),
        scratch_types=[
            pltpu.VMEM((RPW,), jnp.int32),
            pltpu.VMEM((GROUP, D), jnp.float32),
            pltpu.SemaphoreType.DMA,
        ],
        compiler_params=pltpu.CompilerParams(use_tc_tiling_on_sc=False),
    )
    def gk(x_hbm, tab_hbm, out_hbm, idx_v, buf_v, sem):
        wid = lax.axis_index("s") * 2 + lax.axis_index("c")
        base = wid * RPW
        pltpu.sync_copy(x_hbm.at[pl.ds(base, RPW)], idx_v)

        lanes = lax.iota(jnp.int32, 16)
        offs = [((r * 16 + lanes) % NF) * V for r in range(NPER)]

        def add_offs(m, carry):
            s0 = m * PERIOD
            for r in range(NPER):
                sl = pl.ds(s0 + r * 16, 16)
                idx_v[sl] = idx_v[sl] + offs[r]
            return carry

        lax.fori_loop(0, RPW // PERIOD, add_offs, 0)

        def group(g, carry):
            row0 = g * GROUP
            cps = [
                pltpu.async_copy(
                    tab_hbm.at[idx_v.at[pl.ds(row0 + k * CHUNK, CHUNK)]],
                    buf_v.at[pl.ds(k * CHUNK, CHUNK)],
                    sem,
                )
                for k in range(NBUF)
            ]
            for cp in cps:
                cp.wait()
            pltpu.sync_copy(buf_v, out_hbm.at[pl.ds(base + row0, GROUP)])
            return carry

        lax.fori_loop(0, NGROUP, group, 0)

    return gk(xf, tf)


def _mlp_body(emb_ref, w1_ref, b1_ref, g1_ref, be1_ref, w2_ref, b2_ref,
              g2_ref, be2_ref, w3_ref, b3_ref, out_ref,
              h1_ref, h2_ref, acc1_ref, acc2_ref):
    p = pl.program_id(0)
    t = pl.program_id(1)

    @pl.when(jnp.logical_and(p == 0, t == 0))
    def _():
        acc1_ref[...] = jnp.zeros_like(acc1_ref)
        acc2_ref[...] = jnp.zeros_like(acc2_ref)

    @pl.when(p == 0)
    def _():
        h = jnp.dot(emb_ref[...], w1_ref[...], preferred_element_type=jnp.float32)
        h = jnp.maximum(h + b1_ref[...], 0.0)
        h1_ref[pl.ds(t * BT, BT), :] = h
        acc1_ref[0:1, :] += jnp.sum(h, axis=0, keepdims=True)
        acc1_ref[1:2, :] += jnp.sum(h * h, axis=0, keepdims=True)

    @pl.when(jnp.logical_and(p == 1, t == 0))
    def _():
        mean = acc1_ref[0:1, :] * (1.0 / B)
        var = acc1_ref[1:2, :] * (1.0 / B) - mean * mean
        sc = g1_ref[...] * lax.rsqrt(var + EPS)
        acc1_ref[2:3, :] = sc
        acc1_ref[3:4, :] = be1_ref[...] - mean * sc

    @pl.when(p == 1)
    def _():
        h1n = h1_ref[pl.ds(t * BT, BT), :] * acc1_ref[2:3, :] + acc1_ref[3:4, :]
        h = jnp.dot(h1n, w2_ref[...], preferred_element_type=jnp.float32)
        h = jnp.maximum(h + b2_ref[...], 0.0)
        h2_ref[pl.ds(t * BT, BT), :] = h
        acc2_ref[0:1, :] += jnp.sum(h, axis=0, keepdims=True)
        acc2_ref[1:2, :] += jnp.sum(h * h, axis=0, keepdims=True)

    @pl.when(jnp.logical_and(p == 2, t == 0))
    def _():
        mean = acc2_ref[0:1, :] * (1.0 / B)
        var = acc2_ref[1:2, :] * (1.0 / B) - mean * mean
        sc = g2_ref[...] * lax.rsqrt(var + EPS)
        acc2_ref[2:3, :] = sc
        acc2_ref[3:4, :] = be2_ref[...] - mean * sc

    @pl.when(p == 2)
    def _():
        h2n = h2_ref[pl.ds(t * BT, BT), :] * acc2_ref[2:3, :] + acc2_ref[3:4, :]
        o = jnp.sum(h2n * w3_ref[...], axis=1, keepdims=True) + b3_ref[...]
        out_ref[...] = o


def _mlp(emb, W1, b1, g1, be1, W2, b2, g2, be2, w3row, b3, interpret=False):
    full = lambda shape: pl.BlockSpec(shape, lambda p, t: (0, 0))
    return pl.pallas_call(
        _mlp_body,
        grid=(3, T),
        in_specs=[
            pl.BlockSpec((BT, EM), lambda p, t: (jnp.where(p == 0, t, 0), 0)),
            full((EM, H1)), full((1, H1)), full((1, H1)), full((1, H1)),
            full((H1, H2)), full((1, H2)), full((1, H2)), full((1, H2)),
            full((1, H2)), full((1, 1)),
        ],
        out_specs=pl.BlockSpec((BT, 1), lambda p, t: (jnp.where(p == 2, t, 0), 0)),
        out_shape=jax.ShapeDtypeStruct((B, 1), jnp.float32),
        scratch_shapes=[
            pltpu.VMEM((B, H1), jnp.float32),
            pltpu.VMEM((B, H2), jnp.float32),
            pltpu.VMEM((8, H1), jnp.float32),
            pltpu.VMEM((8, H2), jnp.float32),
        ],
        compiler_params=pltpu.CompilerParams(
            dimension_semantics=("arbitrary", "arbitrary")),
        interpret=interpret,
    )(emb, W1, b1, g1, be1, W2, b2, g2, be2, w3row, b3)


def kernel(x, tables, W1, b1, gamma1, beta1, W2, b2, gamma2, beta2, W3, b3):
    xf = x.reshape(ROWS).astype(jnp.int32)
    tf = tables.reshape(NF * V, D)
    rows = _sc_gather(xf, tf)
    emb = rows.reshape(B, EM)
    out = _mlp(emb, W1,
               b1.reshape(1, H1), gamma1.reshape(1, H1), beta1.reshape(1, H1),
               W2, b2.reshape(1, H2), gamma2.reshape(1, H2), beta2.reshape(1, H2),
               W3.reshape(1, H2), b3.reshape(1, 1))
    return out[:, 0]


# speedup vs baseline: 5.2621x; 2.6563x over previous
import functools

import jax
import jax.numpy as jnp
from jax import lax
from jax.experimental import pallas as pl
from jax.experimental.pallas import tpu as pltpu
from jax.experimental.pallas import tpu_sc as plsc

V = 100000
NF = 26
D = 32
B = 16384
EM = NF * D
H1 = 128
H2 = 64
NW = 32
CPW = EM // NW
EPS = 1e-5
BT = 1024
T = B // BT


def _sc_gather_t(xtf, tabT):
    mesh = plsc.VectorSubcoreMesh(core_axis_name="c", subcore_axis_name="s")

    @functools.partial(
        pl.kernel,
        mesh=mesh,
        out_type=jax.ShapeDtypeStruct((EM, B), jnp.float32),
        scratch_types=[
            pltpu.VMEM((B,), jnp.float32),
            pltpu.VMEM((V,), jnp.float32),
        ],
        compiler_params=pltpu.CompilerParams(use_tc_tiling_on_sc=True,
                                             needs_layout_passes=False),
    )
    def gk(x_hbm, tab_hbm, out_hbm, val_v, trow_v):
        wid = lax.axis_index("s") * 2 + lax.axis_index("c")

        def chan(i, carry):
            c = wid * CPW + i
            f = c // D
            pltpu.sync_copy(x_hbm.at[f], val_v)
            pltpu.sync_copy(tab_hbm.at[c], trow_v)

            def vec(k, inner):
                sl = pl.ds(k * 16, 16)
                iv = plsc.bitcast(val_v[sl], jnp.int32)
                val_v[sl] = plsc.load_gather(trow_v, [iv])
                return inner

            lax.fori_loop(0, B // 16, vec, 0)
            pltpu.sync_copy(val_v, out_hbm.at[c])
            return carry

        lax.fori_loop(0, CPW, chan, 0)

    return gk(xtf, tabT)


def _mlp_body(emb_ref, w1_ref, b1_ref, g1_ref, be1_ref, w2_ref, b2_ref,
              g2_ref, be2_ref, w3_ref, b3_ref, out_ref,
              h1_ref, h2_ref, acc1_ref, acc2_ref):
    p = pl.program_id(0)
    t = pl.program_id(1)

    @pl.when(jnp.logical_and(p == 0, t == 0))
    def _():
        acc1_ref[...] = jnp.zeros_like(acc1_ref)
        acc2_ref[...] = jnp.zeros_like(acc2_ref)

    @pl.when(p == 0)
    def _():
        h = jnp.dot(w1_ref[...], emb_ref[...], preferred_element_type=jnp.float32)
        h = jnp.maximum(h + b1_ref[...], 0.0)
        h1_ref[:, pl.ds(t * BT, BT)] = h
        acc1_ref[:, 0:1] += jnp.sum(h, axis=1, keepdims=True)
        acc1_ref[:, 1:2] += jnp.sum(h * h, axis=1, keepdims=True)

    @pl.when(jnp.logical_and(p == 1, t == 0))
    def _():
        mean = acc1_ref[:, 0:1] * (1.0 / B)
        var = acc1_ref[:, 1:2] * (1.0 / B) - mean * mean
        sc = g1_ref[...] * lax.rsqrt(var + EPS)
        acc1_ref[:, 2:3] = sc
        acc1_ref[:, 3:4] = be1_ref[...] - mean * sc

    @pl.when(p == 1)
    def _():
        h1n = h1_ref[:, pl.ds(t * BT, BT)] * acc1_ref[:, 2:3] + acc1_ref[:, 3:4]
        h = jnp.dot(w2_ref[...], h1n, preferred_element_type=jnp.float32)
        h = jnp.maximum(h + b2_ref[...], 0.0)
        h2_ref[:, pl.ds(t * BT, BT)] = h
        acc2_ref[:, 0:1] += jnp.sum(h, axis=1, keepdims=True)
        acc2_ref[:, 1:2] += jnp.sum(h * h, axis=1, keepdims=True)

    @pl.when(jnp.logical_and(p == 2, t == 0))
    def _():
        mean = acc2_ref[:, 0:1] * (1.0 / B)
        var = acc2_ref[:, 1:2] * (1.0 / B) - mean * mean
        sc = g2_ref[...] * lax.rsqrt(var + EPS)
        acc2_ref[:, 2:3] = sc
        acc2_ref[:, 3:4] = be2_ref[...] - mean * sc

    @pl.when(p == 2)
    def _():
        h2n = h2_ref[:, pl.ds(t * BT, BT)] * acc2_ref[:, 2:3] + acc2_ref[:, 3:4]
        o = jnp.sum(h2n * w3_ref[...], axis=0, keepdims=True) + b3_ref[...]
        out_ref[...] = o


def _mlp_t(embT, W1T, b1, g1, be1, W2T, b2, g2, be2, w3, b3, interpret=False):
    full = lambda shape: pl.BlockSpec(shape, lambda p, t: (0, 0))
    return pl.pallas_call(
        _mlp_body,
        grid=(3, T),
        in_specs=[
            pl.BlockSpec((EM, BT), lambda p, t: (0, jnp.where(p == 0, t, 0))),
            full((H1, EM)), full((H1, 1)), full((H1, 1)), full((H1, 1)),
            full((H2, H1)), full((H2, 1)), full((H2, 1)), full((H2, 1)),
            full((H2, 1)), full((1, 1)),
        ],
        out_specs=pl.BlockSpec((1, BT), lambda p, t: (0, jnp.where(p == 2, t, 0))),
        out_shape=jax.ShapeDtypeStruct((1, B), jnp.float32),
        scratch_shapes=[
            pltpu.VMEM((H1, B), jnp.float32),
            pltpu.VMEM((H2, B), jnp.float32),
            pltpu.VMEM((H1, 8), jnp.float32),
            pltpu.VMEM((H2, 8), jnp.float32),
        ],
        compiler_params=pltpu.CompilerParams(
            dimension_semantics=("arbitrary", "arbitrary")),
        interpret=interpret,
    )(embT, W1T, b1, g1, be1, W2T, b2, g2, be2, w3, b3)


def kernel(x, tables, W1, b1, gamma1, beta1, W2, b2, gamma2, beta2, W3, b3):
    xtf = lax.bitcast_convert_type(x.T.astype(jnp.int32), jnp.float32)
    tabT = tables.transpose(0, 2, 1).reshape(EM, V)
    embT = _sc_gather_t(xtf, tabT)
    outT = _mlp_t(embT, W1.T,
                  b1.reshape(H1, 1), gamma1.reshape(H1, 1), beta1.reshape(H1, 1),
                  W2.T, b2.reshape(H2, 1), gamma2.reshape(H2, 1), beta2.reshape(H2, 1),
                  W3, b3.reshape(1, 1))
    return outT[0]


# speedup vs baseline: 6.5884x; 1.2521x over previous
import functools

import jax
import jax.numpy as jnp
from jax import lax
from jax.experimental import pallas as pl
from jax.experimental.pallas import tpu as pltpu
from jax.experimental.pallas import tpu_sc as plsc

V = 100000
NF = 26
D = 32
B = 16384
EM = NF * D
H1 = 128
H2 = 64
NW = 32
CPW = EM // NW
OC = 4096
EPS = 1e-5
BT = 1024
T = B // BT


def _sc_gather_t(xtf, tabT):
    mesh = plsc.VectorSubcoreMesh(core_axis_name="c", subcore_axis_name="s")

    @functools.partial(
        pl.kernel,
        mesh=mesh,
        out_type=jax.ShapeDtypeStruct((EM, B), jnp.float32),
        scratch_types=[
            pltpu.VMEM((B,), jnp.float32),
            pltpu.VMEM((V,), jnp.float32),
            pltpu.VMEM((OC,), jnp.float32),
            pltpu.VMEM((OC,), jnp.float32),
            pltpu.SemaphoreType.DMA,
        ],
        compiler_params=pltpu.CompilerParams(use_tc_tiling_on_sc=True,
                                             needs_layout_passes=False),
    )
    def gk(x_hbm, tab_hbm, out_hbm, idx_v, trow_v, ova, ovb, sem):
        wid = lax.axis_index("s") * 2 + lax.axis_index("c")

        def chan(i, carry):
            c = wid * CPW + i
            f = c // D

            @pl.when(jnp.logical_or(i == 0, f != (c - 1) // D))
            def _():
                pltpu.sync_copy(x_hbm.at[f], idx_v)

            pltpu.sync_copy(tab_hbm.at[c], trow_v)

            def gather_chunk(q):
                buf = ova if q % 2 == 0 else ovb

                def vec(k, inner):
                    for u in range(4):
                        s = k * 64 + u * 16
                        iv = plsc.bitcast(idx_v[pl.ds(q * OC + s, 16)], jnp.int32)
                        buf[pl.ds(s, 16)] = plsc.load_gather(trow_v, [iv])
                    return inner

                lax.fori_loop(0, OC // 64, vec, 0)
                return pltpu.async_copy(
                    buf, out_hbm.at[c, pl.ds(q * OC, OC)], sem)

            hs = [None, None]
            for q in range(B // OC):
                if hs[q % 2] is not None:
                    hs[q % 2].wait()
                hs[q % 2] = gather_chunk(q)
            for h in hs:
                h.wait()
            return carry

        lax.fori_loop(0, CPW, chan, 0)

    return gk(xtf, tabT)


def _mlp_body(emb_ref, w1_ref, b1_ref, g1_ref, be1_ref, w2_ref, b2_ref,
              g2_ref, be2_ref, w3_ref, b3_ref, out_ref,
              h1_ref, h2_ref, acc1_ref, acc2_ref):
    p = pl.program_id(0)
    t = pl.program_id(1)

    @pl.when(jnp.logical_and(p == 0, t == 0))
    def _():
        acc1_ref[...] = jnp.zeros_like(acc1_ref)
        acc2_ref[...] = jnp.zeros_like(acc2_ref)

    @pl.when(p == 0)
    def _():
        h = jnp.dot(w1_ref[...], emb_ref[...], preferred_element_type=jnp.float32)
        h = jnp.maximum(h + b1_ref[...], 0.0)
        h1_ref[:, pl.ds(t * BT, BT)] = h
        acc1_ref[:, 0:1] += jnp.sum(h, axis=1, keepdims=True)
        acc1_ref[:, 1:2] += jnp.sum(h * h, axis=1, keepdims=True)

    @pl.when(jnp.logical_and(p == 1, t == 0))
    def _():
        mean = acc1_ref[:, 0:1] * (1.0 / B)
        var = acc1_ref[:, 1:2] * (1.0 / B) - mean * mean
        sc = g1_ref[...] * lax.rsqrt(var + EPS)
        acc1_ref[:, 2:3] = sc
        acc1_ref[:, 3:4] = be1_ref[...] - mean * sc

    @pl.when(p == 1)
    def _():
        h1n = h1_ref[:, pl.ds(t * BT, BT)] * acc1_ref[:, 2:3] + acc1_ref[:, 3:4]
        h = jnp.dot(w2_ref[...], h1n, preferred_element_type=jnp.float32)
        h = jnp.maximum(h + b2_ref[...], 0.0)
        h2_ref[:, pl.ds(t * BT, BT)] = h
        acc2_ref[:, 0:1] += jnp.sum(h, axis=1, keepdims=True)
        acc2_ref[:, 1:2] += jnp.sum(h * h, axis=1, keepdims=True)

    @pl.when(jnp.logical_and(p == 2, t == 0))
    def _():
        mean = acc2_ref[:, 0:1] * (1.0 / B)
        var = acc2_ref[:, 1:2] * (1.0 / B) - mean * mean
        sc = g2_ref[...] * lax.rsqrt(var + EPS)
        acc2_ref[:, 2:3] = sc
        acc2_ref[:, 3:4] = be2_ref[...] - mean * sc

    @pl.when(p == 2)
    def _():
        h2n = h2_ref[:, pl.ds(t * BT, BT)] * acc2_ref[:, 2:3] + acc2_ref[:, 3:4]
        o = jnp.sum(h2n * w3_ref[...], axis=0, keepdims=True) + b3_ref[...]
        out_ref[...] = o


def _mlp_t(embT, W1T, b1, g1, be1, W2T, b2, g2, be2, w3, b3, interpret=False):
    full = lambda shape: pl.BlockSpec(shape, lambda p, t: (0, 0))
    return pl.pallas_call(
        _mlp_body,
        grid=(3, T),
        in_specs=[
            pl.BlockSpec((EM, BT), lambda p, t: (0, jnp.where(p == 0, t, 0))),
            full((H1, EM)), full((H1, 1)), full((H1, 1)), full((H1, 1)),
            full((H2, H1)), full((H2, 1)), full((H2, 1)), full((H2, 1)),
            full((H2, 1)), full((1, 1)),
        ],
        out_specs=pl.BlockSpec((1, BT), lambda p, t: (0, jnp.where(p == 2, t, 0))),
        out_shape=jax.ShapeDtypeStruct((1, B), jnp.float32),
        scratch_shapes=[
            pltpu.VMEM((H1, B), jnp.float32),
            pltpu.VMEM((H2, B), jnp.float32),
            pltpu.VMEM((H1, 8), jnp.float32),
            pltpu.VMEM((H2, 8), jnp.float32),
        ],
        compiler_params=pltpu.CompilerParams(
            dimension_semantics=("arbitrary", "arbitrary")),
        interpret=interpret,
    )(embT, W1T, b1, g1, be1, W2T, b2, g2, be2, w3, b3)


def kernel(x, tables, W1, b1, gamma1, beta1, W2, b2, gamma2, beta2, W3, b3):
    xtf = lax.bitcast_convert_type(x.T.astype(jnp.int32), jnp.float32)
    tabT = tables.transpose(0, 2, 1).reshape(EM, V)
    embT = _sc_gather_t(xtf, tabT)
    outT = _mlp_t(embT, W1.T,
                  b1.reshape(H1, 1), gamma1.reshape(H1, 1), beta1.reshape(H1, 1),
                  W2.T, b2.reshape(H2, 1), gamma2.reshape(H2, 1), beta2.reshape(H2, 1),
                  W3, b3.reshape(1, 1))
    return outT[0]
